# baseline (speedup 1.0000x reference)
import functools

import jax
import jax.numpy as jnp
from jax import lax
from jax.experimental import pallas as pl
from jax.experimental.pallas import tpu as pltpu
from jax.experimental.pallas import tpu_sc as plsc

N, E, D, H = 10000, 320000, 128, 128
NPAD = 10240
NC, NS = 2, 16
NW = NC * NS
NCH = 80
EPAD = NW * NCH * 128
RPT = NPAD // NS
MB = NPAD // 128
NCHT = NW * NCH
CPW = NCHT // NS
PH = 32

_sc_mesh = plsc.VectorSubcoreMesh(
    core_axis_name="c", subcore_axis_name="s", num_cores=NC, num_subcores=NS)


def _zero_1d(ref, n):
    def body(i, _):
        ref[pl.ds(i * 16, 16)] = jnp.zeros((16,), jnp.float32)
        return 0
    lax.fori_loop(0, n // 16, body, 0)


def _tile_reduce_and_write(acc_v, blk_v, red_v, shared, out_slice, s):
    pltpu.sync_copy(acc_v, shared.at[s])
    plsc.subcore_barrier()
    pltpu.sync_copy(shared.at[:, pl.ds(s * RPT, RPT)], blk_v)

    def body(i, _):
        v = blk_v[0, pl.ds(i * 16, 16)]
        for k in range(1, NS):
            v = v + blk_v[k, pl.ds(i * 16, 16)]
        red_v[pl.ds(i * 16, 16)] = v
        return 0
    lax.fori_loop(0, RPT // 16, body, 0)
    pltpu.sync_copy(red_v, out_slice)


@functools.partial(
    pl.kernel, mesh=_sc_mesh,
    compiler_params=pltpu.CompilerParams(needs_layout_passes=False),
    out_type=jax.ShapeDtypeStruct((NC, NPAD), jnp.float32),
    scratch_types=[
        pltpu.VMEM((NCH, 128), jnp.int32),
        pltpu.VMEM((NPAD,), jnp.float32),
        pltpu.VMEM((NS, RPT), jnp.float32),
        pltpu.VMEM((RPT,), jnp.float32),
        pltpu.VMEM_SHARED((NS, NPAD), jnp.float32),
    ],
)
def _deg_kernel(dst_hbm, out_hbm, dst_v, acc_v, blk_v, red_v, shared):
    c = lax.axis_index("c")
    s = lax.axis_index("s")
    w = c * NS + s
    pltpu.sync_copy(dst_hbm.at[pl.ds(w * NCH, NCH)], dst_v)
    _zero_1d(acc_v, NPAD)
    ones = jnp.ones((16,), jnp.float32)

    def body(j, _):
        for k in range(8):
            d16 = dst_v[j, pl.ds(k * 16, 16)]
            plsc.addupdate_scatter(acc_v, [d16], ones)
        return 0
    lax.fori_loop(0, NCH, body, 0)
    _tile_reduce_and_write(acc_v, blk_v, red_v, shared,
                           out_hbm.at[c, pl.ds(s * RPT, RPT)], s)


@functools.partial(
    pl.kernel, mesh=_sc_mesh,
    compiler_params=pltpu.CompilerParams(needs_layout_passes=False),
    out_type=jax.ShapeDtypeStruct((NPAD, D), jnp.float32),
    scratch_types=[
        pltpu.VMEM((PH, 128), jnp.int32),
        pltpu.VMEM((PH, 128), jnp.int32),
        pltpu.VMEM((128, D), jnp.float32),
        pltpu.VMEM((128, D), jnp.float32),
        pltpu.VMEM_SHARED((NPAD, D), jnp.float32),
        pltpu.SemaphoreType.DMA,
        pltpu.SemaphoreType.DMA,
    ],
)
def _prop_kernel(g_hbm, src_hbm, dst_hbm, out_hbm,
                 src_v, dst_v, bufa, bufb, shared, sema, semb):
    c = lax.axis_index("c")
    s = lax.axis_index("s")

    @pl.when(c == 0)
    def _():
        pltpu.sync_copy(g_hbm.at[pl.ds(s * RPT, RPT)],
                        shared.at[pl.ds(s * RPT, RPT)])
        plsc.subcore_barrier()

        def phase(p, _):
            row0 = s * CPW + p * PH
            pltpu.sync_copy(src_hbm.at[pl.ds(row0, PH)], src_v)
            pltpu.sync_copy(dst_hbm.at[pl.ds(row0, PH)], dst_v)

            def body(j, _):
                ca = pltpu.async_copy(g_hbm.at[src_v.at[2 * j]], bufa, sema)
                cb = pltpu.async_copy(g_hbm.at[src_v.at[2 * j + 1]], bufb,
                                      semb)
                ca.wait()
                pltpu.sync_copy(bufa, shared.at[dst_v.at[2 * j]], add=True)
                cb.wait()
                pltpu.sync_copy(bufb, shared.at[dst_v.at[2 * j + 1]],
                                add=True)
                return 0
            lax.fori_loop(0, PH // 2, body, 0)
            return 0
        lax.fori_loop(0, CPW // PH, phase, 0)
        plsc.subcore_barrier()
        pltpu.sync_copy(shared.at[pl.ds(s * RPT, RPT)],
                        out_hbm.at[pl.ds(s * RPT, RPT)])


@functools.partial(
    pl.kernel, mesh=_sc_mesh,
    compiler_params=pltpu.CompilerParams(needs_layout_passes=False),
    out_type=jax.ShapeDtypeStruct((NC, NPAD), jnp.float32),
    scratch_types=[
        pltpu.VMEM((NPAD,), jnp.float32),
        pltpu.VMEM((NCH, 128), jnp.int32),
        pltpu.VMEM((NCH, 128), jnp.int32),
        pltpu.VMEM((NPAD,), jnp.float32),
        pltpu.VMEM((NS, RPT), jnp.float32),
        pltpu.VMEM((RPT,), jnp.float32),
        pltpu.VMEM_SHARED((NS, NPAD), jnp.float32),
    ],
)
def _sprop_kernel(zs_hbm, src_hbm, dst_hbm, out_hbm,
                  zs_v, src_v, dst_v, acc_v, blk_v, red_v, shared):
    c = lax.axis_index("c")
    s = lax.axis_index("s")
    w = c * NS + s
    pltpu.sync_copy(zs_hbm, zs_v)
    pltpu.sync_copy(src_hbm.at[pl.ds(w * NCH, NCH)], src_v)
    pltpu.sync_copy(dst_hbm.at[pl.ds(w * NCH, NCH)], dst_v)
    _zero_1d(acc_v, NPAD)

    def body(j, _):
        for k in range(8):
            s16 = src_v[j, pl.ds(k * 16, 16)]
            d16 = dst_v[j, pl.ds(k * 16, 16)]
            vals = plsc.load_gather(zs_v, [s16])
            plsc.addupdate_scatter(acc_v, [d16], vals)
        return 0
    lax.fori_loop(0, NCH, body, 0)
    _tile_reduce_and_write(acc_v, blk_v, red_v, shared,
                           out_hbm.at[c, pl.ds(s * RPT, RPT)], s)


def _mm1_body(x_ref, w1_ref, p0_ref, p1_ref, g_ref, dinv_ref):
    deg = p0_ref[...] + p1_ref[...] + 1.0
    dinv = lax.rsqrt(deg)
    h = jnp.dot(x_ref[...], w1_ref[...], preferred_element_type=jnp.float32)
    g_ref[...] = h * dinv
    dinv_ref[...] = dinv


_mm1 = pl.pallas_call(
    _mm1_body,
    grid=(MB,),
    in_specs=[
        pl.BlockSpec((128, D), lambda i: (i, 0)),
        pl.BlockSpec((D, H), lambda i: (0, 0)),
        pl.BlockSpec((128, 1), lambda i: (i, 0)),
        pl.BlockSpec((128, 1), lambda i: (i, 0)),
    ],
    out_specs=[
        pl.BlockSpec((128, H), lambda i: (i, 0)),
        pl.BlockSpec((128, 1), lambda i: (i, 0)),
    ],
    out_shape=[
        jax.ShapeDtypeStruct((NPAD, H), jnp.float32),
        jax.ShapeDtypeStruct((NPAD, 1), jnp.float32),
    ],
)


def _mid_body(a0_ref, dinv_ref, b1_ref, w2_ref, zs_ref):
    i = pl.program_id(0)
    dinv = dinv_ref[...]
    out1 = dinv * a0_ref[...] + b1_ref[...]
    r = jnp.maximum(out1, 0.0)
    z = jnp.dot(r, w2_ref[...], preferred_element_type=jnp.float32)
    rows = i * 128 + lax.broadcasted_iota(jnp.int32, (128, 1), 0)
    zs_ref[...] = jnp.where(rows < N, dinv * z, 0.0)


_mid = pl.pallas_call(
    _mid_body,
    grid=(MB,),
    in_specs=[
        pl.BlockSpec((128, H), lambda i: (i, 0)),
        pl.BlockSpec((128, 1), lambda i: (i, 0)),
        pl.BlockSpec((1, H), lambda i: (0, 0)),
        pl.BlockSpec((H, 1), lambda i: (0, 0)),
    ],
    out_specs=pl.BlockSpec((128, 1), lambda i: (i, 0)),
    out_shape=jax.ShapeDtypeStruct((NPAD, 1), jnp.float32),
)


def _fin_body(q0_ref, q1_ref, zs_ref, dinv_ref, b2_ref, out_ref):
    out_ref[...] = (dinv_ref[...] * (q0_ref[...] + q1_ref[...] + zs_ref[...])
                    + b2_ref[...])


_fin = pl.pallas_call(
    _fin_body,
    in_specs=[
        pl.BlockSpec((MB, 128), lambda: (0, 0)),
        pl.BlockSpec((MB, 128), lambda: (0, 0)),
        pl.BlockSpec((MB, 128), lambda: (0, 0)),
        pl.BlockSpec((MB, 128), lambda: (0, 0)),
        pl.BlockSpec((1, 1), lambda: (0, 0)),
    ],
    out_specs=pl.BlockSpec((MB, 128), lambda: (0, 0)),
    out_shape=jax.ShapeDtypeStruct((MB, 128), jnp.float32),
)


def kernel(x, edge_index, W1, b1, W2, b2):
    xp = jnp.pad(x, ((0, NPAD - N), (0, 0)))
    pad = jnp.full((EPAD - E,), NPAD - 1, dtype=jnp.int32)
    srcp = jnp.concatenate([edge_index[0], pad]).reshape(NCHT, 128)
    dstp = jnp.concatenate([edge_index[1], pad]).reshape(NCHT, 128)

    degp = _deg_kernel(dstp)
    p0 = degp[0].reshape(NPAD, 1)
    p1 = degp[1].reshape(NPAD, 1)
    g, dinv = _mm1(xp, W1, p0, p1)
    acc = _prop_kernel(g, srcp, dstp)
    zs = _mid(acc, dinv, b1.reshape(1, H), W2)
    q = _sprop_kernel(zs.reshape(NPAD), srcp, dstp)
    fin = _fin(q[0].reshape(MB, 128), q[1].reshape(MB, 128),
               zs.reshape(MB, 128), dinv.reshape(MB, 128),
               b2.reshape(1, 1))
    return fin.reshape(NPAD)[:N].reshape(N, 1)


# speedup vs baseline: 1.0714x; 1.0714x over previous
import functools

import jax
import jax.numpy as jnp
from jax import lax
from jax.experimental import pallas as pl
from jax.experimental.pallas import tpu as pltpu
from jax.experimental.pallas import tpu_sc as plsc

N, E, D, H = 10000, 320000, 128, 128
NPAD = 10240
NC, NS = 2, 16
NW = NC * NS
NCH = 80
EPAD = NW * NCH * 128
RPT = NPAD // NS
MB = NPAD // 128
NCHT = NW * NCH
CPW = NCHT // NS
PH = 32

_sc_mesh = plsc.VectorSubcoreMesh(
    core_axis_name="c", subcore_axis_name="s", num_cores=NC, num_subcores=NS)


def _zero_1d(ref, n):
    def body(i, _):
        ref[pl.ds(i * 16, 16)] = jnp.zeros((16,), jnp.float32)
        return 0
    lax.fori_loop(0, n // 16, body, 0)


def _tile_reduce_and_write(acc_v, blk_v, red_v, shared, out_slice, s):
    pltpu.sync_copy(acc_v, shared.at[s])
    plsc.subcore_barrier()
    pltpu.sync_copy(shared.at[:, pl.ds(s * RPT, RPT)], blk_v)

    def body(i, _):
        v = blk_v[0, pl.ds(i * 16, 16)]
        for k in range(1, NS):
            v = v + blk_v[k, pl.ds(i * 16, 16)]
        red_v[pl.ds(i * 16, 16)] = v
        return 0
    lax.fori_loop(0, RPT // 16, body, 0)
    pltpu.sync_copy(red_v, out_slice)


@functools.partial(
    pl.kernel, mesh=_sc_mesh,
    compiler_params=pltpu.CompilerParams(needs_layout_passes=False),
    out_type=jax.ShapeDtypeStruct((NC, NPAD), jnp.float32),
    scratch_types=[
        pltpu.VMEM((NCH, 128), jnp.int32),
        pltpu.VMEM((NPAD,), jnp.float32),
        pltpu.VMEM((NS, RPT), jnp.float32),
        pltpu.VMEM((RPT,), jnp.float32),
        pltpu.VMEM_SHARED((NS, NPAD), jnp.float32),
    ],
)
def _deg_kernel(dst_hbm, out_hbm, dst_v, acc_v, blk_v, red_v, shared):
    c = lax.axis_index("c")
    s = lax.axis_index("s")
    w = c * NS + s
    pltpu.sync_copy(dst_hbm.at[pl.ds(w * NCH, NCH)], dst_v)
    _zero_1d(acc_v, NPAD)
    ones = jnp.ones((16,), jnp.float32)

    def body(j, _):
        for k in range(8):
            d16 = dst_v[j, pl.ds(k * 16, 16)]
            plsc.addupdate_scatter(acc_v, [d16], ones)
        return 0
    lax.fori_loop(0, NCH, body, 0)
    _tile_reduce_and_write(acc_v, blk_v, red_v, shared,
                           out_hbm.at[c, pl.ds(s * RPT, RPT)], s)


@functools.partial(
    pl.kernel, mesh=_sc_mesh,
    compiler_params=pltpu.CompilerParams(needs_layout_passes=False),
    out_type=jax.ShapeDtypeStruct((NPAD, D), jnp.float32),
    scratch_types=[
        pltpu.VMEM((PH, 128), jnp.int32),
        pltpu.VMEM((PH, 128), jnp.int32),
        pltpu.VMEM((128, D), jnp.float32),
        pltpu.VMEM((128, D), jnp.float32),
        pltpu.VMEM_SHARED((NPAD, D), jnp.float32),
        pltpu.SemaphoreType.DMA,
        pltpu.SemaphoreType.DMA,
        pltpu.SemaphoreType.DMA,
        pltpu.SemaphoreType.DMA,
    ],
)
def _prop_kernel(g_hbm, src_hbm, dst_hbm, out_hbm,
                 src_v, dst_v, bufa, bufb, shared, ga, gb, sa, sb):
    c = lax.axis_index("c")
    s = lax.axis_index("s")


    def wait_dma(buf, sem):
        pltpu.make_async_copy(g_hbm.at[pl.ds(0, 128)], buf, sem).wait()

    @pl.when(c == 0)
    def _():
        pltpu.sync_copy(g_hbm.at[pl.ds(s * RPT, RPT)],
                        shared.at[pl.ds(s * RPT, RPT)])
        plsc.subcore_barrier()

        def phase(p, _):
            row0 = s * CPW + p * PH
            pltpu.sync_copy(src_hbm.at[pl.ds(row0, PH)], src_v)
            pltpu.sync_copy(dst_hbm.at[pl.ds(row0, PH)], dst_v)

            pltpu.async_copy(g_hbm.at[src_v.at[0]], bufa, ga)
            wait_dma(bufa, ga)
            pltpu.async_copy(g_hbm.at[src_v.at[1]], bufb, gb)
            pltpu.async_copy(bufa, shared.at[dst_v.at[0]], sa, add=True)

            def body(k, _):
                jb = 2 * k + 1
                ja = 2 * k + 2
                wait_dma(bufb, gb)
                scb = pltpu.async_copy(bufb, shared.at[dst_v.at[jb]], sb,
                                       add=True)
                wait_dma(bufa, sa)
                pltpu.async_copy(g_hbm.at[src_v.at[ja]], bufa, ga)
                wait_dma(bufa, ga)
                pltpu.async_copy(bufa, shared.at[dst_v.at[ja]], sa, add=True)
                scb.wait()
                pltpu.async_copy(g_hbm.at[src_v.at[ja + 1]], bufb, gb)
                return 0
            lax.fori_loop(0, PH // 2 - 1, body, 0)

            wait_dma(bufb, gb)
            pltpu.async_copy(bufb, shared.at[dst_v.at[PH - 1]], sb, add=True)
            wait_dma(bufa, sa)
            wait_dma(bufb, sb)
            return 0
        lax.fori_loop(0, CPW // PH, phase, 0)
        plsc.subcore_barrier()
        pltpu.sync_copy(shared.at[pl.ds(s * RPT, RPT)],
                        out_hbm.at[pl.ds(s * RPT, RPT)])


@functools.partial(
    pl.kernel, mesh=_sc_mesh,
    compiler_params=pltpu.CompilerParams(needs_layout_passes=False),
    out_type=jax.ShapeDtypeStruct((NC, NPAD), jnp.float32),
    scratch_types=[
        pltpu.VMEM((NPAD,), jnp.float32),
        pltpu.VMEM((NCH, 128), jnp.int32),
        pltpu.VMEM((NCH, 128), jnp.int32),
        pltpu.VMEM((NPAD,), jnp.float32),
        pltpu.VMEM((NS, RPT), jnp.float32),
        pltpu.VMEM((RPT,), jnp.float32),
        pltpu.VMEM_SHARED((NS, NPAD), jnp.float32),
    ],
)
def _sprop_kernel(zs_hbm, src_hbm, dst_hbm, out_hbm,
                  zs_v, src_v, dst_v, acc_v, blk_v, red_v, shared):
    c = lax.axis_index("c")
    s = lax.axis_index("s")
    w = c * NS + s
    pltpu.sync_copy(zs_hbm, zs_v)
    pltpu.sync_copy(src_hbm.at[pl.ds(w * NCH, NCH)], src_v)
    pltpu.sync_copy(dst_hbm.at[pl.ds(w * NCH, NCH)], dst_v)
    _zero_1d(acc_v, NPAD)

    def body(j, _):
        for k in range(8):
            s16 = src_v[j, pl.ds(k * 16, 16)]
            d16 = dst_v[j, pl.ds(k * 16, 16)]
            vals = plsc.load_gather(zs_v, [s16])
            plsc.addupdate_scatter(acc_v, [d16], vals)
        return 0
    lax.fori_loop(0, NCH, body, 0)
    _tile_reduce_and_write(acc_v, blk_v, red_v, shared,
                           out_hbm.at[c, pl.ds(s * RPT, RPT)], s)


def _mm1_body(x_ref, w1_ref, p0_ref, p1_ref, g_ref, dinv_ref):
    deg = p0_ref[...] + p1_ref[...] + 1.0
    dinv = lax.rsqrt(deg)
    h = jnp.dot(x_ref[...], w1_ref[...], preferred_element_type=jnp.float32)
    g_ref[...] = h * dinv
    dinv_ref[...] = dinv


_mm1 = pl.pallas_call(
    _mm1_body,
    grid=(MB,),
    in_specs=[
        pl.BlockSpec((128, D), lambda i: (i, 0)),
        pl.BlockSpec((D, H), lambda i: (0, 0)),
        pl.BlockSpec((128, 1), lambda i: (i, 0)),
        pl.BlockSpec((128, 1), lambda i: (i, 0)),
    ],
    out_specs=[
        pl.BlockSpec((128, H), lambda i: (i, 0)),
        pl.BlockSpec((128, 1), lambda i: (i, 0)),
    ],
    out_shape=[
        jax.ShapeDtypeStruct((NPAD, H), jnp.float32),
        jax.ShapeDtypeStruct((NPAD, 1), jnp.float32),
    ],
)


def _mid_body(a0_ref, dinv_ref, b1_ref, w2_ref, zs_ref):
    i = pl.program_id(0)
    dinv = dinv_ref[...]
    out1 = dinv * a0_ref[...] + b1_ref[...]
    r = jnp.maximum(out1, 0.0)
    z = jnp.dot(r, w2_ref[...], preferred_element_type=jnp.float32)
    rows = i * 128 + lax.broadcasted_iota(jnp.int32, (128, 1), 0)
    zs_ref[...] = jnp.where(rows < N, dinv * z, 0.0)


_mid = pl.pallas_call(
    _mid_body,
    grid=(MB,),
    in_specs=[
        pl.BlockSpec((128, H), lambda i: (i, 0)),
        pl.BlockSpec((128, 1), lambda i: (i, 0)),
        pl.BlockSpec((1, H), lambda i: (0, 0)),
        pl.BlockSpec((H, 1), lambda i: (0, 0)),
    ],
    out_specs=pl.BlockSpec((128, 1), lambda i: (i, 0)),
    out_shape=jax.ShapeDtypeStruct((NPAD, 1), jnp.float32),
)


def _fin_body(q0_ref, q1_ref, zs_ref, dinv_ref, b2_ref, out_ref):
    out_ref[...] = (dinv_ref[...] * (q0_ref[...] + q1_ref[...] + zs_ref[...])
                    + b2_ref[...])


_fin = pl.pallas_call(
    _fin_body,
    in_specs=[
        pl.BlockSpec((MB, 128), lambda: (0, 0)),
        pl.BlockSpec((MB, 128), lambda: (0, 0)),
        pl.BlockSpec((MB, 128), lambda: (0, 0)),
        pl.BlockSpec((MB, 128), lambda: (0, 0)),
        pl.BlockSpec((1, 1), lambda: (0, 0)),
    ],
    out_specs=pl.BlockSpec((MB, 128), lambda: (0, 0)),
    out_shape=jax.ShapeDtypeStruct((MB, 128), jnp.float32),
)


def kernel(x, edge_index, W1, b1, W2, b2):
    xp = jnp.pad(x, ((0, NPAD - N), (0, 0)))
    pad = jnp.full((EPAD - E,), NPAD - 1, dtype=jnp.int32)
    srcp = jnp.concatenate([edge_index[0], pad]).reshape(NCHT, 128)
    dstp = jnp.concatenate([edge_index[1], pad]).reshape(NCHT, 128)

    degp = _deg_kernel(dstp)
    p0 = degp[0].reshape(NPAD, 1)
    p1 = degp[1].reshape(NPAD, 1)
    g, dinv = _mm1(xp, W1, p0, p1)
    acc = _prop_kernel(g, srcp, dstp)
    zs = _mid(acc, dinv, b1.reshape(1, H), W2)
    q = _sprop_kernel(zs.reshape(NPAD), srcp, dstp)
    fin = _fin(q[0].reshape(MB, 128), q[1].reshape(MB, 128),
               zs.reshape(MB, 128), dinv.reshape(MB, 128),
               b2.reshape(1, 1))
    return fin.reshape(NPAD)[:N].reshape(N, 1)


# speedup vs baseline: 1.8795x; 1.7543x over previous
import functools

import jax
import jax.numpy as jnp
from jax import lax
from jax.experimental import pallas as pl
from jax.experimental.pallas import tpu as pltpu
from jax.experimental.pallas import tpu_sc as plsc

N, E, D, H = 10000, 320000, 128, 128
NPAD = 10240
NC, NS = 2, 16
NW = NC * NS
NCH = 80
EPAD = NW * NCH * 128
RPT = NPAD // NS
MB = NPAD // 128
NCHT = NW * NCH
CPW = NCHT // NS
PH = 32

_sc_mesh = plsc.VectorSubcoreMesh(
    core_axis_name="c", subcore_axis_name="s", num_cores=NC, num_subcores=NS)


def _zero_1d(ref, n):
    def body(i, _):
        ref[pl.ds(i * 16, 16)] = jnp.zeros((16,), jnp.float32)
        return 0
    lax.fori_loop(0, n // 16, body, 0)


def _tile_reduce_and_write(acc_v, blk_v, red_v, shared, out_slice, s):
    pltpu.sync_copy(acc_v, shared.at[s])
    plsc.subcore_barrier()
    pltpu.sync_copy(shared.at[:, pl.ds(s * RPT, RPT)], blk_v)

    def body(i, _):
        v = blk_v[0, pl.ds(i * 16, 16)]
        for k in range(1, NS):
            v = v + blk_v[k, pl.ds(i * 16, 16)]
        red_v[pl.ds(i * 16, 16)] = v
        return 0
    lax.fori_loop(0, RPT // 16, body, 0)
    pltpu.sync_copy(red_v, out_slice)


@functools.partial(
    pl.kernel, mesh=_sc_mesh,
    compiler_params=pltpu.CompilerParams(needs_layout_passes=False),
    out_type=jax.ShapeDtypeStruct((NC, NPAD), jnp.float32),
    scratch_types=[
        pltpu.VMEM((NCH, 128), jnp.int32),
        pltpu.VMEM((NPAD,), jnp.float32),
        pltpu.VMEM((NS, RPT), jnp.float32),
        pltpu.VMEM((RPT,), jnp.float32),
        pltpu.VMEM_SHARED((NS, NPAD), jnp.float32),
    ],
)
def _deg_kernel(dst_hbm, out_hbm, dst_v, acc_v, blk_v, red_v, shared):
    c = lax.axis_index("c")
    s = lax.axis_index("s")
    w = c * NS + s
    pltpu.sync_copy(dst_hbm.at[pl.ds(w * NCH, NCH)], dst_v)
    _zero_1d(acc_v, NPAD)
    ones = jnp.ones((16,), jnp.float32)

    def body(j, _):
        for k in range(8):
            d16 = dst_v[j, pl.ds(k * 16, 16)]
            plsc.addupdate_scatter(acc_v, [d16], ones)
        return 0
    lax.fori_loop(0, NCH, body, 0)
    _tile_reduce_and_write(acc_v, blk_v, red_v, shared,
                           out_hbm.at[c, pl.ds(s * RPT, RPT)], s)


@functools.partial(
    pl.kernel, mesh=_sc_mesh,
    compiler_params=pltpu.CompilerParams(needs_layout_passes=False),
    out_type=jax.ShapeDtypeStruct((NPAD, D), jnp.float32),
    scratch_types=[
        pltpu.VMEM((PH, 128), jnp.int32),
        pltpu.VMEM((PH, 128), jnp.int32),
        pltpu.VMEM((128, D), jnp.float32),
        pltpu.VMEM((128, D), jnp.float32),
        pltpu.VMEM_SHARED((NPAD, D), jnp.float32),
        pltpu.SemaphoreType.DMA,
        pltpu.SemaphoreType.DMA,
        pltpu.SemaphoreType.DMA,
        pltpu.SemaphoreType.DMA,
    ],
)
def _prop_kernel(g_hbm, src_hbm, dst_hbm, out_hbm,
                 src_v, dst_v, bufa, bufb, shared, ga, gb, sa, sb):
    c = lax.axis_index("c")
    s = lax.axis_index("s")


    def wait_dma(buf, sem):
        pltpu.make_async_copy(g_hbm.at[pl.ds(0, 128)], buf, sem).wait()

    @pl.when(c == 0)
    def _():
        pltpu.sync_copy(g_hbm.at[pl.ds(s * RPT, RPT)],
                        shared.at[pl.ds(s * RPT, RPT)])
        plsc.subcore_barrier()

        def phase(p, _):
            row0 = s * CPW + p * PH
            pltpu.sync_copy(src_hbm.at[pl.ds(row0, PH)], src_v)
            pltpu.sync_copy(dst_hbm.at[pl.ds(row0, PH)], dst_v)

            pltpu.async_copy(g_hbm.at[src_v.at[0]], bufa, ga)
            wait_dma(bufa, ga)
            pltpu.async_copy(g_hbm.at[src_v.at[1]], bufb, gb)
            pltpu.async_copy(bufa, shared.at[dst_v.at[0]], sa, add=True)

            def body(k, _):
                jb = 2 * k + 1
                ja = 2 * k + 2
                wait_dma(bufb, gb)
                scb = pltpu.async_copy(bufb, shared.at[dst_v.at[jb]], sb,
                                       add=True)
                wait_dma(bufa, sa)
                pltpu.async_copy(g_hbm.at[src_v.at[ja]], bufa, ga)
                wait_dma(bufa, ga)
                pltpu.async_copy(bufa, shared.at[dst_v.at[ja]], sa, add=True)
                scb.wait()
                pltpu.async_copy(g_hbm.at[src_v.at[ja + 1]], bufb, gb)
                return 0
            lax.fori_loop(0, PH // 2 - 1, body, 0)

            wait_dma(bufb, gb)
            pltpu.async_copy(bufb, shared.at[dst_v.at[PH - 1]], sb, add=True)
            wait_dma(bufa, sa)
            wait_dma(bufb, sb)
            return 0
        lax.fori_loop(0, CPW // PH, phase, 0)
        plsc.subcore_barrier()
        pltpu.sync_copy(shared.at[pl.ds(s * RPT, RPT)],
                        out_hbm.at[pl.ds(s * RPT, RPT)])


@functools.partial(
    pl.kernel, mesh=_sc_mesh,
    compiler_params=pltpu.CompilerParams(needs_layout_passes=False),
    out_type=jax.ShapeDtypeStruct((NC, NPAD), jnp.float32),
    scratch_types=[
        pltpu.VMEM((NPAD,), jnp.float32),
        pltpu.VMEM((NCH, 128), jnp.int32),
        pltpu.VMEM((NCH, 128), jnp.int32),
        pltpu.VMEM((NPAD,), jnp.float32),
        pltpu.VMEM((NS, RPT), jnp.float32),
        pltpu.VMEM((RPT,), jnp.float32),
        pltpu.VMEM_SHARED((NS, NPAD), jnp.float32),
    ],
)
def _sprop_kernel(zs_hbm, src_hbm, dst_hbm, out_hbm,
                  zs_v, src_v, dst_v, acc_v, blk_v, red_v, shared):
    c = lax.axis_index("c")
    s = lax.axis_index("s")
    w = c * NS + s
    pltpu.sync_copy(zs_hbm, zs_v)
    pltpu.sync_copy(src_hbm.at[pl.ds(w * NCH, NCH)], src_v)
    pltpu.sync_copy(dst_hbm.at[pl.ds(w * NCH, NCH)], dst_v)
    _zero_1d(acc_v, NPAD)

    def body(j, _):
        for k in range(8):
            s16 = src_v[j, pl.ds(k * 16, 16)]
            d16 = dst_v[j, pl.ds(k * 16, 16)]
            vals = plsc.load_gather(zs_v, [s16])
            plsc.addupdate_scatter(acc_v, [d16], vals)
        return 0
    lax.fori_loop(0, NCH, body, 0)
    _tile_reduce_and_write(acc_v, blk_v, red_v, shared,
                           out_hbm.at[c, pl.ds(s * RPT, RPT)], s)


def _mm1_body(x_ref, w1_ref, p0_ref, p1_ref, g_ref, dinv_ref):
    deg = p0_ref[...] + p1_ref[...] + 1.0
    dinv = lax.rsqrt(deg)
    h = jnp.dot(x_ref[...], w1_ref[...], preferred_element_type=jnp.float32)
    g_ref[...] = h * dinv
    dinv_ref[...] = dinv


_mm1 = pl.pallas_call(
    _mm1_body,
    grid=(MB,),
    in_specs=[
        pl.BlockSpec((128, D), lambda i: (i, 0)),
        pl.BlockSpec((D, H), lambda i: (0, 0)),
        pl.BlockSpec((128, 1), lambda i: (i, 0)),
        pl.BlockSpec((128, 1), lambda i: (i, 0)),
    ],
    out_specs=[
        pl.BlockSpec((128, H), lambda i: (i, 0)),
        pl.BlockSpec((128, 1), lambda i: (i, 0)),
    ],
    out_shape=[
        jax.ShapeDtypeStruct((NPAD, H), jnp.float32),
        jax.ShapeDtypeStruct((NPAD, 1), jnp.float32),
    ],
)


def _mid_body(a0_ref, dinv_ref, b1_ref, w2_ref, zs_ref):
    i = pl.program_id(0)
    dinv = dinv_ref[...]
    out1 = dinv * a0_ref[...] + b1_ref[...]
    r = jnp.maximum(out1, 0.0)
    z = jnp.dot(r, w2_ref[...], preferred_element_type=jnp.float32)
    rows = i * 128 + lax.broadcasted_iota(jnp.int32, (128, 1), 0)
    zs_ref[...] = jnp.where(rows < N, dinv * z, 0.0)


_mid = pl.pallas_call(
    _mid_body,
    grid=(MB,),
    in_specs=[
        pl.BlockSpec((128, H), lambda i: (i, 0)),
        pl.BlockSpec((128, 1), lambda i: (i, 0)),
        pl.BlockSpec((1, H), lambda i: (0, 0)),
        pl.BlockSpec((H, 1), lambda i: (0, 0)),
    ],
    out_specs=pl.BlockSpec((128, 1), lambda i: (i, 0)),
    out_shape=jax.ShapeDtypeStruct((NPAD, 1), jnp.float32),
)


def _fin_body(q0_ref, q1_ref, zs_ref, dinv_ref, b2_ref, out_ref):
    out_ref[...] = (dinv_ref[...] * (q0_ref[...] + q1_ref[...] + zs_ref[...])
                    + b2_ref[...])


_fin = pl.pallas_call(
    _fin_body,
    in_specs=[
        pl.BlockSpec((MB, 128), lambda: (0, 0)),
        pl.BlockSpec((MB, 128), lambda: (0, 0)),
        pl.BlockSpec((MB, 128), lambda: (0, 0)),
        pl.BlockSpec((MB, 128), lambda: (0, 0)),
        pl.BlockSpec((1, 1), lambda: (0, 0)),
    ],
    out_specs=pl.BlockSpec((MB, 128), lambda: (0, 0)),
    out_shape=jax.ShapeDtypeStruct((MB, 128), jnp.float32),
)


def kernel(x, edge_index, W1, b1, W2, b2):
    xp = jnp.pad(x, ((0, NPAD - N), (0, 0)))
    pad = (jnp.arange(EPAD - E, dtype=jnp.int32) % (NPAD - N)) + N
    srcp = jnp.concatenate([edge_index[0], pad]).reshape(NCHT, 128)
    dstp = jnp.concatenate([edge_index[1], pad]).reshape(NCHT, 128)

    degp = _deg_kernel(dstp)
    p0 = degp[0].reshape(NPAD, 1)
    p1 = degp[1].reshape(NPAD, 1)
    g, dinv = _mm1(xp, W1, p0, p1)
    acc = _prop_kernel(g, srcp, dstp)
    zs = _mid(acc, dinv, b1.reshape(1, H), W2)
    q = _sprop_kernel(zs.reshape(NPAD), srcp, dstp)
    fin = _fin(q[0].reshape(MB, 128), q[1].reshape(MB, 128),
               zs.reshape(MB, 128), dinv.reshape(MB, 128),
               b2.reshape(1, 1))
    return fin.reshape(NPAD)[:N].reshape(N, 1)


# speedup vs baseline: 2.5567x; 1.3603x over previous
import functools

import jax
import jax.numpy as jnp
from jax import lax
from jax.experimental import pallas as pl
from jax.experimental.pallas import tpu as pltpu
from jax.experimental.pallas import tpu_sc as plsc

N, E, D, H = 10000, 320000, 128, 128
NPAD = 10240
NC, NS = 2, 16
NW = NC * NS
NCH = 80
EPAD = NW * NCH * 128
RPT = NPAD // NS
MB = NPAD // 128
NCHT = NW * NCH
CPW = NCHT // NW
PH = 40

_sc_mesh = plsc.VectorSubcoreMesh(
    core_axis_name="c", subcore_axis_name="s", num_cores=NC, num_subcores=NS)


def _zero_1d(ref, n):
    def body(i, _):
        ref[pl.ds(i * 16, 16)] = jnp.zeros((16,), jnp.float32)
        return 0
    lax.fori_loop(0, n // 16, body, 0)


def _tile_reduce_and_write(acc_v, blk_v, red_v, shared, out_slice, s):
    pltpu.sync_copy(acc_v, shared.at[s])
    plsc.subcore_barrier()
    pltpu.sync_copy(shared.at[:, pl.ds(s * RPT, RPT)], blk_v)

    def body(i, _):
        v = blk_v[0, pl.ds(i * 16, 16)]
        for k in range(1, NS):
            v = v + blk_v[k, pl.ds(i * 16, 16)]
        red_v[pl.ds(i * 16, 16)] = v
        return 0
    lax.fori_loop(0, RPT // 16, body, 0)
    pltpu.sync_copy(red_v, out_slice)


@functools.partial(
    pl.kernel, mesh=_sc_mesh,
    compiler_params=pltpu.CompilerParams(needs_layout_passes=False),
    out_type=jax.ShapeDtypeStruct((NC, NPAD), jnp.float32),
    scratch_types=[
        pltpu.VMEM((NCH, 128), jnp.int32),
        pltpu.VMEM((NPAD,), jnp.float32),
        pltpu.VMEM((NS, RPT), jnp.float32),
        pltpu.VMEM((RPT,), jnp.float32),
        pltpu.VMEM_SHARED((NS, NPAD), jnp.float32),
    ],
)
def _deg_kernel(dst_hbm, out_hbm, dst_v, acc_v, blk_v, red_v, shared):
    c = lax.axis_index("c")
    s = lax.axis_index("s")
    w = c * NS + s
    pltpu.sync_copy(dst_hbm.at[pl.ds(w * NCH, NCH)], dst_v)
    _zero_1d(acc_v, NPAD)
    ones = jnp.ones((16,), jnp.float32)

    def body(j, _):
        for k in range(8):
            d16 = dst_v[j, pl.ds(k * 16, 16)]
            plsc.addupdate_scatter(acc_v, [d16], ones)
        return 0
    lax.fori_loop(0, NCH, body, 0)
    _tile_reduce_and_write(acc_v, blk_v, red_v, shared,
                           out_hbm.at[c, pl.ds(s * RPT, RPT)], s)


@functools.partial(
    pl.kernel, mesh=_sc_mesh,
    compiler_params=pltpu.CompilerParams(needs_layout_passes=False),
    out_type=jax.ShapeDtypeStruct((NC, NPAD, D), jnp.float32),
    scratch_types=[
        pltpu.VMEM((PH, 128), jnp.int32),
        pltpu.VMEM((PH, 128), jnp.int32),
        pltpu.VMEM((128, D), jnp.float32),
        pltpu.VMEM((128, D), jnp.float32),
        pltpu.VMEM_SHARED((NPAD, D), jnp.float32),
        pltpu.SemaphoreType.DMA,
        pltpu.SemaphoreType.DMA,
        pltpu.SemaphoreType.DMA,
        pltpu.SemaphoreType.DMA,
    ],
)
def _prop_kernel(g_hbm, src_hbm, dst_hbm, out_hbm,
                 src_v, dst_v, bufa, bufb, shared, ga, gb, sa, sb):
    c = lax.axis_index("c")
    s = lax.axis_index("s")
    w = c * NS + s


    def wait_dma(buf, sem):
        pltpu.make_async_copy(g_hbm.at[pl.ds(0, 128)], buf, sem).wait()

    pltpu.sync_copy(g_hbm.at[pl.ds(s * RPT, RPT)],
                    shared.at[pl.ds(s * RPT, RPT)])
    plsc.subcore_barrier()

    def phase(p, _):
        row0 = w * CPW + p * PH
        pltpu.sync_copy(src_hbm.at[pl.ds(row0, PH)], src_v)
        pltpu.sync_copy(dst_hbm.at[pl.ds(row0, PH)], dst_v)

        pltpu.async_copy(g_hbm.at[src_v.at[0]], bufa, ga)
        wait_dma(bufa, ga)
        pltpu.async_copy(g_hbm.at[src_v.at[1]], bufb, gb)
        pltpu.async_copy(bufa, shared.at[dst_v.at[0]], sa, add=True)

        def body(k, _):
            jb = 2 * k + 1
            ja = 2 * k + 2
            wait_dma(bufb, gb)
            scb = pltpu.async_copy(bufb, shared.at[dst_v.at[jb]], sb,
                                   add=True)
            wait_dma(bufa, sa)
            pltpu.async_copy(g_hbm.at[src_v.at[ja]], bufa, ga)
            wait_dma(bufa, ga)
            pltpu.async_copy(bufa, shared.at[dst_v.at[ja]], sa, add=True)
            scb.wait()
            pltpu.async_copy(g_hbm.at[src_v.at[ja + 1]], bufb, gb)
            return 0
        lax.fori_loop(0, PH // 2 - 1, body, 0)

        wait_dma(bufb, gb)
        pltpu.async_copy(bufb, shared.at[dst_v.at[PH - 1]], sb, add=True)
        wait_dma(bufa, sa)
        wait_dma(bufb, sb)
        return 0
    lax.fori_loop(0, CPW // PH, phase, 0)
    plsc.subcore_barrier()
    pltpu.sync_copy(shared.at[pl.ds(s * RPT, RPT)],
                    out_hbm.at[c, pl.ds(s * RPT, RPT)])


@functools.partial(
    pl.kernel, mesh=_sc_mesh,
    compiler_params=pltpu.CompilerParams(needs_layout_passes=False),
    out_type=jax.ShapeDtypeStruct((NC, NPAD), jnp.float32),
    scratch_types=[
        pltpu.VMEM((NPAD,), jnp.float32),
        pltpu.VMEM((NCH, 128), jnp.int32),
        pltpu.VMEM((NCH, 128), jnp.int32),
        pltpu.VMEM((NPAD,), jnp.float32),
        pltpu.VMEM((NS, RPT), jnp.float32),
        pltpu.VMEM((RPT,), jnp.float32),
        pltpu.VMEM_SHARED((NS, NPAD), jnp.float32),
    ],
)
def _sprop_kernel(zs_hbm, src_hbm, dst_hbm, out_hbm,
                  zs_v, src_v, dst_v, acc_v, blk_v, red_v, shared):
    c = lax.axis_index("c")
    s = lax.axis_index("s")
    w = c * NS + s
    pltpu.sync_copy(zs_hbm, zs_v)
    pltpu.sync_copy(src_hbm.at[pl.ds(w * NCH, NCH)], src_v)
    pltpu.sync_copy(dst_hbm.at[pl.ds(w * NCH, NCH)], dst_v)
    _zero_1d(acc_v, NPAD)

    def body(j, _):
        for k in range(8):
            s16 = src_v[j, pl.ds(k * 16, 16)]
            d16 = dst_v[j, pl.ds(k * 16, 16)]
            vals = plsc.load_gather(zs_v, [s16])
            plsc.addupdate_scatter(acc_v, [d16], vals)
        return 0
    lax.fori_loop(0, NCH, body, 0)
    _tile_reduce_and_write(acc_v, blk_v, red_v, shared,
                           out_hbm.at[c, pl.ds(s * RPT, RPT)], s)


def _mm1_body(x_ref, w1_ref, p0_ref, p1_ref, g_ref, dinv_ref):
    deg = p0_ref[...] + p1_ref[...] + 1.0
    dinv = lax.rsqrt(deg)
    h = jnp.dot(x_ref[...], w1_ref[...], preferred_element_type=jnp.float32)
    g_ref[...] = h * dinv
    dinv_ref[...] = dinv


_mm1 = pl.pallas_call(
    _mm1_body,
    grid=(MB,),
    in_specs=[
        pl.BlockSpec((128, D), lambda i: (i, 0)),
        pl.BlockSpec((D, H), lambda i: (0, 0)),
        pl.BlockSpec((128, 1), lambda i: (i, 0)),
        pl.BlockSpec((128, 1), lambda i: (i, 0)),
    ],
    out_specs=[
        pl.BlockSpec((128, H), lambda i: (i, 0)),
        pl.BlockSpec((128, 1), lambda i: (i, 0)),
    ],
    out_shape=[
        jax.ShapeDtypeStruct((NPAD, H), jnp.float32),
        jax.ShapeDtypeStruct((NPAD, 1), jnp.float32),
    ],
)


def _mid_body(a0_ref, a1_ref, g_ref, dinv_ref, b1_ref, w2_ref, zs_ref):
    i = pl.program_id(0)
    dinv = dinv_ref[...]
    acc = a0_ref[0] + a1_ref[0] - g_ref[...]
    out1 = dinv * acc + b1_ref[...]
    r = jnp.maximum(out1, 0.0)
    z = jnp.dot(r, w2_ref[...], preferred_element_type=jnp.float32)
    rows = i * 128 + lax.broadcasted_iota(jnp.int32, (128, 1), 0)
    zs_ref[...] = jnp.where(rows < N, dinv * z, 0.0)


_mid = pl.pallas_call(
    _mid_body,
    grid=(MB,),
    in_specs=[
        pl.BlockSpec((1, 128, H), lambda i: (0, i, 0)),
        pl.BlockSpec((1, 128, H), lambda i: (1, i, 0)),
        pl.BlockSpec((128, H), lambda i: (i, 0)),
        pl.BlockSpec((128, 1), lambda i: (i, 0)),
        pl.BlockSpec((1, H), lambda i: (0, 0)),
        pl.BlockSpec((H, 1), lambda i: (0, 0)),
    ],
    out_specs=pl.BlockSpec((128, 1), lambda i: (i, 0)),
    out_shape=jax.ShapeDtypeStruct((NPAD, 1), jnp.float32),
)


def _fin_body(q0_ref, q1_ref, zs_ref, dinv_ref, b2_ref, out_ref):
    out_ref[...] = (dinv_ref[...] * (q0_ref[...] + q1_ref[...] + zs_ref[...])
                    + b2_ref[...])


_fin = pl.pallas_call(
    _fin_body,
    in_specs=[
        pl.BlockSpec((MB, 128), lambda: (0, 0)),
        pl.BlockSpec((MB, 128), lambda: (0, 0)),
        pl.BlockSpec((MB, 128), lambda: (0, 0)),
        pl.BlockSpec((MB, 128), lambda: (0, 0)),
        pl.BlockSpec((1, 1), lambda: (0, 0)),
    ],
    out_specs=pl.BlockSpec((MB, 128), lambda: (0, 0)),
    out_shape=jax.ShapeDtypeStruct((MB, 128), jnp.float32),
)


def kernel(x, edge_index, W1, b1, W2, b2):
    xp = jnp.pad(x, ((0, NPAD - N), (0, 0)))
    pad = (jnp.arange(EPAD - E, dtype=jnp.int32) % (NPAD - N)) + N
    srcp = jnp.concatenate([edge_index[0], pad]).reshape(NCHT, 128)
    dstp = jnp.concatenate([edge_index[1], pad]).reshape(NCHT, 128)

    degp = _deg_kernel(dstp)
    p0 = degp[0].reshape(NPAD, 1)
    p1 = degp[1].reshape(NPAD, 1)
    g, dinv = _mm1(xp, W1, p0, p1)
    acc = _prop_kernel(g, srcp, dstp)
    zs = _mid(acc, acc, g, dinv, b1.reshape(1, H), W2)
    q = _sprop_kernel(zs.reshape(NPAD), srcp, dstp)
    fin = _fin(q[0].reshape(MB, 128), q[1].reshape(MB, 128),
               zs.reshape(MB, 128), dinv.reshape(MB, 128),
               b2.reshape(1, 1))
    return fin.reshape(NPAD)[:N].reshape(N, 1)
